# Initial kernel scaffold; baseline (speedup 1.0000x reference)
import jax
import jax.numpy as jnp
from jax.experimental import pallas as pl


def kernel(prefix, emb, W1, b1, W2, b2):
    raise NotImplementedError("write your pallas kernel here")


# speedup vs baseline: 1.1256x; 1.1256x over previous
import jax
import jax.numpy as jnp
from jax.experimental import pallas as pl
from jax.experimental.pallas import tpu as pltpu

PRE_SEQ_LEN = 128
HIDDEN = 1024
OUT_DIM = 24 * 2 * 1024
TOKENS = 8 * 128
BLK = 1024


def _body(prefix_ref, emb_ref, w1_ref, b1_ref, w2_ref, b2_ref, out_ref,
          h_ref, oh_ref):
    j = pl.program_id(0)

    @pl.when(j == 0)
    def _init():
        h_ref[...] = jnp.tanh(
            jnp.dot(emb_ref[...], w1_ref[...],
                    preferred_element_type=jnp.float32) + b1_ref[...])
        row_ids = jax.lax.broadcasted_iota(jnp.int32, (TOKENS, PRE_SEQ_LEN), 1)
        oh_ref[...] = (prefix_ref[...] == row_ids).astype(jnp.float32)

    t = jnp.dot(h_ref[...], w2_ref[...], preferred_element_type=jnp.float32)
    out_ref[...] = jnp.dot(oh_ref[...], t,
                           preferred_element_type=jnp.float32) + b2_ref[...]


def kernel(prefix, emb, W1, b1, W2, b2):
    prefix2d = prefix.reshape(TOKENS, 1).astype(jnp.int32)
    b1r = b1.reshape(1, HIDDEN)
    b2r = b2.reshape(1, OUT_DIM)
    grid = (OUT_DIM // BLK,)
    out = pl.pallas_call(
        _body,
        grid=grid,
        in_specs=[
            pl.BlockSpec((TOKENS, 1), lambda j: (0, 0)),
            pl.BlockSpec((PRE_SEQ_LEN, HIDDEN), lambda j: (0, 0)),
            pl.BlockSpec((HIDDEN, HIDDEN), lambda j: (0, 0)),
            pl.BlockSpec((1, HIDDEN), lambda j: (0, 0)),
            pl.BlockSpec((HIDDEN, BLK), lambda j: (0, j)),
            pl.BlockSpec((1, BLK), lambda j: (0, j)),
        ],
        out_specs=pl.BlockSpec((TOKENS, BLK), lambda j: (0, j)),
        out_shape=jax.ShapeDtypeStruct((TOKENS, OUT_DIM), jnp.float32),
        scratch_shapes=[
            pltpu.VMEM((PRE_SEQ_LEN, HIDDEN), jnp.float32),
            pltpu.VMEM((TOKENS, PRE_SEQ_LEN), jnp.float32),
        ],
        compiler_params=pltpu.CompilerParams(
            dimension_semantics=("arbitrary",),
        ),
    )(prefix2d, emb, W1, b1r, W2, b2r)
    return out.reshape(prefix.shape[0], prefix.shape[1], OUT_DIM)


# speedup vs baseline: 1.1276x; 1.0018x over previous
import jax
import jax.numpy as jnp
from jax.experimental import pallas as pl
from jax.experimental.pallas import tpu as pltpu

PRE_SEQ_LEN = 128
HIDDEN = 1024
OUT_DIM = 24 * 2 * 1024
TOKENS = 8 * 128
BLK = 1024


def _body(prefix_ref, emb_ref, w1_ref, b1_ref, w2_ref, b2_ref, out_ref,
          h_ref, oh_ref):
    j = pl.program_id(0)

    @pl.when(j == 0)
    def _init():
        h_ref[...] = jnp.tanh(
            jnp.dot(emb_ref[...], w1_ref[...],
                    preferred_element_type=jnp.float32) + b1_ref[...]
        ).astype(jnp.bfloat16)
        row_ids = jax.lax.broadcasted_iota(jnp.int32, (TOKENS, PRE_SEQ_LEN), 1)
        oh_ref[...] = (prefix_ref[...] == row_ids).astype(jnp.bfloat16)

    t = jnp.dot(h_ref[...], w2_ref[...].astype(jnp.bfloat16),
                preferred_element_type=jnp.float32)
    out_ref[...] = jnp.dot(oh_ref[...], t.astype(jnp.bfloat16),
                           preferred_element_type=jnp.float32) + b2_ref[...]


def kernel(prefix, emb, W1, b1, W2, b2):
    prefix2d = prefix.reshape(TOKENS, 1).astype(jnp.int32)
    b1r = b1.reshape(1, HIDDEN)
    b2r = b2.reshape(1, OUT_DIM)
    grid = (OUT_DIM // BLK,)
    out = pl.pallas_call(
        _body,
        grid=grid,
        in_specs=[
            pl.BlockSpec((TOKENS, 1), lambda j: (0, 0)),
            pl.BlockSpec((PRE_SEQ_LEN, HIDDEN), lambda j: (0, 0)),
            pl.BlockSpec((HIDDEN, HIDDEN), lambda j: (0, 0)),
            pl.BlockSpec((1, HIDDEN), lambda j: (0, 0)),
            pl.BlockSpec((HIDDEN, BLK), lambda j: (0, j)),
            pl.BlockSpec((1, BLK), lambda j: (0, j)),
        ],
        out_specs=pl.BlockSpec((TOKENS, BLK), lambda j: (0, j)),
        out_shape=jax.ShapeDtypeStruct((TOKENS, OUT_DIM), jnp.float32),
        scratch_shapes=[
            pltpu.VMEM((PRE_SEQ_LEN, HIDDEN), jnp.bfloat16),
            pltpu.VMEM((TOKENS, PRE_SEQ_LEN), jnp.bfloat16),
        ],
        compiler_params=pltpu.CompilerParams(
            dimension_semantics=("arbitrary",),
        ),
    )(prefix2d, emb, W1, b1r, W2, b2r)
    return out.reshape(prefix.shape[0], prefix.shape[1], OUT_DIM)


# speedup vs baseline: 1.1602x; 1.0289x over previous
import jax
import jax.numpy as jnp
from jax.experimental import pallas as pl
from jax.experimental.pallas import tpu as pltpu

PRE_SEQ_LEN = 128
HIDDEN = 1024
OUT_DIM = 24 * 2 * 1024
TOKENS = 8 * 128
BLK = 2048


def _body(prefix_ref, emb_ref, w1_ref, b1_ref, w2_ref, b2_ref, out_ref,
          h_ref, oh_ref):
    j = pl.program_id(0)

    @pl.when(j == 0)
    def _init():
        h_ref[...] = jnp.tanh(
            jnp.dot(emb_ref[...], w1_ref[...],
                    preferred_element_type=jnp.float32) + b1_ref[...]
        ).astype(jnp.bfloat16)
        row_ids = jax.lax.broadcasted_iota(jnp.int32, (TOKENS, PRE_SEQ_LEN), 1)
        oh_ref[...] = (prefix_ref[...] == row_ids).astype(jnp.bfloat16)

    t = jnp.dot(h_ref[...], w2_ref[...].astype(jnp.bfloat16),
                preferred_element_type=jnp.float32)
    out_ref[...] = jnp.dot(oh_ref[...], t.astype(jnp.bfloat16),
                           preferred_element_type=jnp.float32) + b2_ref[...]


def kernel(prefix, emb, W1, b1, W2, b2):
    prefix2d = prefix.reshape(TOKENS, 1).astype(jnp.int32)
    b1r = b1.reshape(1, HIDDEN)
    b2r = b2.reshape(1, OUT_DIM)
    grid = (OUT_DIM // BLK,)
    out = pl.pallas_call(
        _body,
        grid=grid,
        in_specs=[
            pl.BlockSpec((TOKENS, 1), lambda j: (0, 0)),
            pl.BlockSpec((PRE_SEQ_LEN, HIDDEN), lambda j: (0, 0)),
            pl.BlockSpec((HIDDEN, HIDDEN), lambda j: (0, 0)),
            pl.BlockSpec((1, HIDDEN), lambda j: (0, 0)),
            pl.BlockSpec((HIDDEN, BLK), lambda j: (0, j)),
            pl.BlockSpec((1, BLK), lambda j: (0, j)),
        ],
        out_specs=pl.BlockSpec((TOKENS, BLK), lambda j: (0, j)),
        out_shape=jax.ShapeDtypeStruct((TOKENS, OUT_DIM), jnp.float32),
        scratch_shapes=[
            pltpu.VMEM((PRE_SEQ_LEN, HIDDEN), jnp.bfloat16),
            pltpu.VMEM((TOKENS, PRE_SEQ_LEN), jnp.bfloat16),
        ],
        compiler_params=pltpu.CompilerParams(
            dimension_semantics=("arbitrary",),
        ),
    )(prefix2d, emb, W1, b1r, W2, b2r)
    return out.reshape(prefix.shape[0], prefix.shape[1], OUT_DIM)
